# baseline (speedup 1.0000x reference)
import functools

import jax
import jax.numpy as jnp
from jax import lax
from jax.experimental import pallas as pl
from jax.experimental.pallas import tpu as pltpu
from jax.experimental.pallas import tpu_sc as plsc

H = 1024
E = 8
I = 1024
IS = 2048
N_GROUP = 4
GSIZE = E // N_GROUP
TOPK_GROUP = 2
TOP_K = 2
ROUTED_SCALING = 2.5

T = 2048
TS = 256
NTILES = 24
NS = NTILES * TS
ISC = 512
NISC = IS // ISC
TC = 256
NTC = T // TC

NW = 32
TPW = T // NW

BF = jnp.bfloat16
F32 = jnp.float32


def _router_body(x_ref, wr_ref, bias_ref, causal_ref, s0_ref, s1_ref,
                 w0_ref, w1_ref, te_ref):
    logits = jax.lax.dot_general(
        wr_ref[...], x_ref[...], (((1,), (1,)), ((), ())),
        preferred_element_type=F32)
    scores = jax.nn.sigmoid(logits)
    choice = scores + bias_ref[...]

    ch = [choice[e:e + 1, :] for e in range(E)]
    sc = [scores[e:e + 1, :] for e in range(E)]

    gs = [ch[2 * j] + ch[2 * j + 1] for j in range(N_GROUP)]
    gmask = []
    for j in range(N_GROUP):
        rank = jnp.zeros_like(gs[j])
        for k in range(N_GROUP):
            if k == j:
                continue
            beats = (gs[k] >= gs[j]) if k < j else (gs[k] > gs[j])
            rank = rank + beats.astype(F32)
        gmask.append(rank < float(TOPK_GROUP))

    ms = [jnp.where(gmask[e // GSIZE], ch[e], 0.0) for e in range(E)]

    sel = []
    for e in range(E):
        rank = jnp.zeros_like(ms[e])
        for f in range(E):
            if f == e:
                continue
            beats = (ms[f] >= ms[e]) if f < e else (ms[f] > ms[e])
            rank = rank + beats.astype(F32)
        sel.append((rank < float(TOP_K)).astype(F32))

    w = [sel[e] * sc[e] for e in range(E)]
    denom = w[0]
    for e in range(1, E):
        denom = denom + w[e]
    denom = denom + 1e-20
    comb = [(w[e] / denom) * ROUTED_SCALING for e in range(E)]

    selmat = jnp.concatenate(sel, axis=0)
    posinc = jax.lax.dot_general(
        selmat, causal_ref[...], (((1,), (0,)), ((), ())),
        preferred_element_type=F32)

    cnt = [jnp.sum(sel[e], axis=1, keepdims=True) for e in range(E)]
    caps = [jnp.floor((cnt[e] + (TS - 1)) / TS) * TS for e in range(E)]
    bases = []
    run = jnp.zeros_like(cnt[0])
    for e in range(E):
        bases.append(run)
        run = run + caps[e]

    slot = [bases[e] + posinc[e:e + 1, :] - sel[e] for e in range(E)]

    nb = jnp.ones_like(sel[0])
    s0 = jnp.zeros_like(sel[0])
    s1 = jnp.zeros_like(sel[0])
    w0 = jnp.zeros_like(sel[0])
    w1 = jnp.zeros_like(sel[0])
    for e in range(E):
        fm = sel[e] * nb
        nb = nb * (1.0 - sel[e])
        sm = sel[e] - fm
        s0 = s0 + fm * slot[e]
        s1 = s1 + sm * slot[e]
        w0 = w0 + fm * comb[e]
        w1 = w1 + sm * comb[e]
    s0_ref[...] = s0.astype(jnp.int32)
    s1_ref[...] = s1.astype(jnp.int32)
    w0_ref[...] = w0
    w1_ref[...] = w1

    jt = jax.lax.broadcasted_iota(jnp.int32, (1, NTILES), 1).astype(F32) * TS
    acc = jnp.zeros_like(jt)
    for e in range(E):
        acc = acc + (jt >= (bases[e] + caps[e])).astype(F32)
    te_ref[...] = jnp.minimum(acc, float(E - 1)).astype(jnp.int32)


def _ffn_body(te_ref, xg_ref, gu_ref, d_ref, out_ref, gu_bf, d_bf):
    i = pl.program_id(0)
    iprev = jnp.maximum(i - 1, 0)
    recast = (i == 0) | (te_ref[i] != te_ref[iprev])

    @pl.when(recast)
    def _cast():
        gu_bf[...] = gu_ref[0].astype(BF)
        d_bf[...] = d_ref[0].astype(BF)

    xb = xg_ref[...].astype(BF)
    gate = jax.lax.dot_general(
        xb, gu_bf[:I, :], (((1,), (1,)), ((), ())),
        preferred_element_type=F32)
    up = jax.lax.dot_general(
        xb, gu_bf[I:, :], (((1,), (1,)), ((), ())),
        preferred_element_type=F32)
    h = ((gate * jax.nn.sigmoid(gate)) * up).astype(BF)
    y = jax.lax.dot_general(
        h, d_bf[...], (((1,), (1,)), ((), ())),
        preferred_element_type=F32)
    out_ref[...] = y


def _shared_body(x_ref, gw_ref, uw_ref, dw_ref, out_ref):
    c = pl.program_id(0)
    xb = x_ref[...]
    gate = jax.lax.dot_general(
        xb, gw_ref[...].astype(BF), (((1,), (1,)), ((), ())),
        preferred_element_type=F32)
    up = jax.lax.dot_general(
        xb, uw_ref[...].astype(BF), (((1,), (1,)), ((), ())),
        preferred_element_type=F32)
    h = ((gate * jax.nn.sigmoid(gate)) * up).astype(BF)
    y = jax.lax.dot_general(
        h, dw_ref[...].astype(BF), (((1,), (1,)), ((), ())),
        preferred_element_type=F32)

    @pl.when(c == 0)
    def _init():
        out_ref[...] = y

    @pl.when(c != 0)
    def _acc():
        out_ref[...] = out_ref[...] + y


def _combine_body(shared_ref, g0_ref, g1_ref, w0_ref, w1_ref, out_ref):
    rr = jax.lax.broadcasted_iota(jnp.int32, (TC, TC), 0)
    cc = jax.lax.broadcasted_iota(jnp.int32, (TC, TC), 1)
    ident = (rr == cc).astype(F32)
    w0c = jax.lax.dot_general(
        ident, w0_ref[0], (((1,), (1,)), ((), ())),
        preferred_element_type=F32)
    w1c = jax.lax.dot_general(
        ident, w1_ref[0], (((1,), (1,)), ((), ())),
        preferred_element_type=F32)
    out_ref[...] = (shared_ref[...]
                    + w0c * g0_ref[...].astype(F32)
                    + w1c * g1_ref[...].astype(F32))


def _sc_mesh():
    return plsc.VectorSubcoreMesh(core_axis_name="c", subcore_axis_name="s")


def _sc_scatter(x3, s0m, s1m):
    @functools.partial(
        pl.kernel,
        mesh=_sc_mesh(),
        out_type=jax.ShapeDtypeStruct((NS, H), F32),
        scratch_types=[
            pltpu.VMEM((TPW,), jnp.int32),
            pltpu.VMEM((TPW,), jnp.int32),
            pltpu.VMEM((TPW, H), F32),
            pltpu.SemaphoreType.DMA,
        ],
    )
    def k(x_hbm, s0_hbm, s1_hbm, xg_hbm, idx0, idx1, rows, sem):
        wid = lax.axis_index("s") * 2 + lax.axis_index("c")
        base = wid * TPW
        pltpu.sync_copy(s0_hbm.at[wid], idx0)
        pltpu.sync_copy(s1_hbm.at[wid], idx1)
        pltpu.sync_copy(x_hbm.at[pl.ds(base, TPW)], rows)
        c0 = pltpu.async_copy(rows, xg_hbm.at[idx0], sem)
        c1 = pltpu.async_copy(rows, xg_hbm.at[idx1], sem)
        c0.wait()
        c1.wait()

    return k(x3, s0m, s1m)


def _sc_gather(yg3, s0m, s1m):
    @functools.partial(
        pl.kernel,
        mesh=_sc_mesh(),
        out_type=(jax.ShapeDtypeStruct((T, H), F32),
                  jax.ShapeDtypeStruct((T, H), F32)),
        scratch_types=[
            pltpu.VMEM((TPW,), jnp.int32),
            pltpu.VMEM((TPW,), jnp.int32),
            pltpu.VMEM((TPW // 2, H), F32),
            pltpu.VMEM((TPW // 2, H), F32),
            pltpu.SemaphoreType.DMA,
            pltpu.SemaphoreType.DMA,
        ],
    )
    def k(yg_hbm, s0_hbm, s1_hbm, g0_hbm, g1_hbm, idx0, idx1, rows_a,
          rows_b, sem_a, sem_b):
        wid = lax.axis_index("s") * 2 + lax.axis_index("c")
        base = wid * TPW
        half = TPW // 2
        pltpu.sync_copy(s0_hbm.at[wid], idx0)
        pltpu.sync_copy(s1_hbm.at[wid], idx1)
        steps = [(idx0, g0_hbm, 0), (idx0, g0_hbm, 1),
                 (idx1, g1_hbm, 0), (idx1, g1_hbm, 1)]
        bufs = [(rows_a, sem_a), (rows_b, sem_b)]
        pend = None
        for n, (idx, dst, c) in enumerate(steps):
            rows, sem = bufs[n % 2]
            cp = pltpu.async_copy(yg_hbm.at[idx.at[pl.ds(c * half, half)]],
                                  rows, sem)
            if pend is not None:
                prows, pdst, pc = pend
                pltpu.sync_copy(prows, pdst.at[pl.ds(base + pc * half, half)])
            cp.wait()
            pend = (rows, dst, c)
        prows, pdst, pc = pend
        pltpu.sync_copy(prows, pdst.at[pl.ds(base + pc * half, half)])

    return k(yg3, s0m, s1m)


@jax.jit
def _run(x, router_weight, bias, gate_up_proj, down_proj,
         shared_gate_w, shared_up_w, shared_down_w):
    x_bf = x.astype(BF)
    bias_col = bias.reshape(E, 1)

    causal = jnp.triu(jnp.ones((T, T), F32))
    s0, s1, w0, w1, te = pl.pallas_call(
        _router_body,
        out_shape=(
            jax.ShapeDtypeStruct((1, T), jnp.int32),
            jax.ShapeDtypeStruct((1, T), jnp.int32),
            jax.ShapeDtypeStruct((1, T), F32),
            jax.ShapeDtypeStruct((1, T), F32),
            jax.ShapeDtypeStruct((1, NTILES), jnp.int32),
        ),
    )(x, router_weight, bias_col, causal)

    s0m = s0.reshape(NW, TPW)
    s1m = s1.reshape(NW, TPW)

    xg = _sc_scatter(x, s0m, s1m)

    grid_spec = pltpu.PrefetchScalarGridSpec(
        num_scalar_prefetch=1,
        grid=(NTILES,),
        in_specs=[
            pl.BlockSpec((TS, H), lambda i, te_r: (i, 0)),
            pl.BlockSpec((1, 2 * I, H), lambda i, te_r: (te_r[i], 0, 0)),
            pl.BlockSpec((1, H, I), lambda i, te_r: (te_r[i], 0, 0)),
        ],
        out_specs=pl.BlockSpec((TS, H), lambda i, te_r: (i, 0)),
        scratch_shapes=[
            pltpu.VMEM((2 * I, H), BF),
            pltpu.VMEM((H, I), BF),
        ],
    )
    yg = pl.pallas_call(
        _ffn_body,
        grid_spec=grid_spec,
        out_shape=jax.ShapeDtypeStruct((NS, H), F32),
    )(te.reshape(NTILES), xg, gate_up_proj, down_proj)

    g0, g1 = _sc_gather(yg, s0m, s1m)

    shared = pl.pallas_call(
        _shared_body,
        grid=(NISC,),
        in_specs=[
            pl.BlockSpec((T, H), lambda c: (0, 0)),
            pl.BlockSpec((ISC, H), lambda c: (c, 0)),
            pl.BlockSpec((ISC, H), lambda c: (c, 0)),
            pl.BlockSpec((H, ISC), lambda c: (0, c)),
        ],
        out_specs=pl.BlockSpec((T, H), lambda c: (0, 0)),
        out_shape=jax.ShapeDtypeStruct((T, H), F32),
    )(x_bf, shared_gate_w, shared_up_w, shared_down_w)

    out = pl.pallas_call(
        _combine_body,
        grid=(NTC,),
        in_specs=[
            pl.BlockSpec((TC, H), lambda t: (t, 0)),
            pl.BlockSpec((TC, H), lambda t: (t, 0)),
            pl.BlockSpec((TC, H), lambda t: (t, 0)),
            pl.BlockSpec((1, 1, TC), lambda t: (0, 0, t)),
            pl.BlockSpec((1, 1, TC), lambda t: (0, 0, t)),
        ],
        out_specs=pl.BlockSpec((TC, H), lambda t: (t, 0)),
        out_shape=jax.ShapeDtypeStruct((T, H), F32),
    )(shared, g0, g1, w0.reshape(1, 1, T), w1.reshape(1, 1, T))

    return out


def kernel(hidden_states, router_weight, e_score_correction_bias,
           gate_up_proj, down_proj, shared_gate_w, shared_up_w,
           shared_down_w):
    B, S, Hd = hidden_states.shape
    x = hidden_states.reshape(-1, Hd)
    out = _run(x, router_weight, e_score_correction_bias, gate_up_proj,
               down_proj, shared_gate_w, shared_up_w, shared_down_w)
    return out.reshape(B, S, Hd)


# speedup vs baseline: 1.0785x; 1.0785x over previous
import functools

import jax
import jax.numpy as jnp
from jax import lax
from jax.experimental import pallas as pl
from jax.experimental.pallas import tpu as pltpu
from jax.experimental.pallas import tpu_sc as plsc

H = 1024
E = 8
I = 1024
IS = 2048
N_GROUP = 4
GSIZE = E // N_GROUP
TOPK_GROUP = 2
TOP_K = 2
ROUTED_SCALING = 2.5

T = 2048
TS = 256
NTILES = 24
NS = NTILES * TS
ISC = 512
NISC = IS // ISC
TC = 256
NTC = T // TC

NW = 32
TPW = T // NW

BF = jnp.bfloat16
F32 = jnp.float32


def _router_body(x_ref, wr_ref, bias_ref, s0_ref, s1_ref,
                 w0_ref, w1_ref, te_ref):
    logits = jax.lax.dot_general(
        wr_ref[...], x_ref[...], (((1,), (1,)), ((), ())),
        preferred_element_type=F32)
    scores = jax.nn.sigmoid(logits)
    choice = scores + bias_ref[...]

    ch = [choice[e:e + 1, :] for e in range(E)]
    sc = [scores[e:e + 1, :] for e in range(E)]

    gs = [ch[2 * j] + ch[2 * j + 1] for j in range(N_GROUP)]
    gmask = []
    for j in range(N_GROUP):
        rank = jnp.zeros_like(gs[j])
        for k in range(N_GROUP):
            if k == j:
                continue
            beats = (gs[k] >= gs[j]) if k < j else (gs[k] > gs[j])
            rank = rank + beats.astype(F32)
        gmask.append(rank < float(TOPK_GROUP))

    ms = [jnp.where(gmask[e // GSIZE], ch[e], 0.0) for e in range(E)]

    sel = []
    for e in range(E):
        rank = jnp.zeros_like(ms[e])
        for f in range(E):
            if f == e:
                continue
            beats = (ms[f] >= ms[e]) if f < e else (ms[f] > ms[e])
            rank = rank + beats.astype(F32)
        sel.append((rank < float(TOP_K)).astype(F32))

    w = [sel[e] * sc[e] for e in range(E)]
    denom = w[0]
    for e in range(1, E):
        denom = denom + w[e]
    denom = denom + 1e-20
    comb = [(w[e] / denom) * ROUTED_SCALING for e in range(E)]

    selmat = jnp.concatenate(sel, axis=0)
    r = jax.lax.broadcasted_iota(jnp.int32, (T, T), 0)
    c = jax.lax.broadcasted_iota(jnp.int32, (T, T), 1)
    causal = (r <= c).astype(F32)
    posinc = jax.lax.dot_general(
        selmat, causal, (((1,), (0,)), ((), ())),
        preferred_element_type=F32)

    cnt = [jnp.sum(sel[e], axis=1, keepdims=True) for e in range(E)]
    caps = [jnp.floor((cnt[e] + (TS - 1)) / TS) * TS for e in range(E)]
    bases = []
    run = jnp.zeros_like(cnt[0])
    for e in range(E):
        bases.append(run)
        run = run + caps[e]

    slot = [bases[e] + posinc[e:e + 1, :] - sel[e] for e in range(E)]

    nb = jnp.ones_like(sel[0])
    s0 = jnp.zeros_like(sel[0])
    s1 = jnp.zeros_like(sel[0])
    w0 = jnp.zeros_like(sel[0])
    w1 = jnp.zeros_like(sel[0])
    for e in range(E):
        fm = sel[e] * nb
        nb = nb * (1.0 - sel[e])
        sm = sel[e] - fm
        s0 = s0 + fm * slot[e]
        s1 = s1 + sm * slot[e]
        w0 = w0 + fm * comb[e]
        w1 = w1 + sm * comb[e]
    s0_ref[...] = s0.astype(jnp.int32)
    s1_ref[...] = s1.astype(jnp.int32)
    w0_ref[...] = w0
    w1_ref[...] = w1

    jt = jax.lax.broadcasted_iota(jnp.int32, (1, NTILES), 1).astype(F32) * TS
    acc = jnp.zeros_like(jt)
    for e in range(E):
        acc = acc + (jt >= (bases[e] + caps[e])).astype(F32)
    te_ref[...] = jnp.minimum(acc, float(E - 1)).astype(jnp.int32)


def _ffn_body(te_ref, xg_ref, gu_ref, d_ref, out_ref, gu_bf, d_bf):
    i = pl.program_id(0)
    iprev = jnp.maximum(i - 1, 0)
    recast = (i == 0) | (te_ref[i] != te_ref[iprev])

    @pl.when(recast)
    def _cast():
        gu_bf[...] = gu_ref[0].astype(BF)
        d_bf[...] = d_ref[0].astype(BF)

    xb = xg_ref[...].astype(BF)
    gate = jax.lax.dot_general(
        xb, gu_bf[:I, :], (((1,), (1,)), ((), ())),
        preferred_element_type=F32)
    up = jax.lax.dot_general(
        xb, gu_bf[I:, :], (((1,), (1,)), ((), ())),
        preferred_element_type=F32)
    h = ((gate * jax.nn.sigmoid(gate)) * up).astype(BF)
    y = jax.lax.dot_general(
        h, d_bf[...], (((1,), (1,)), ((), ())),
        preferred_element_type=F32)
    out_ref[...] = y


def _shared_body(x_ref, gw_ref, uw_ref, dw_ref, out_ref):
    c = pl.program_id(0)
    xb = x_ref[...]
    gate = jax.lax.dot_general(
        xb, gw_ref[...].astype(BF), (((1,), (1,)), ((), ())),
        preferred_element_type=F32)
    up = jax.lax.dot_general(
        xb, uw_ref[...].astype(BF), (((1,), (1,)), ((), ())),
        preferred_element_type=F32)
    h = ((gate * jax.nn.sigmoid(gate)) * up).astype(BF)
    y = jax.lax.dot_general(
        h, dw_ref[...].astype(BF), (((1,), (1,)), ((), ())),
        preferred_element_type=F32)

    @pl.when(c == 0)
    def _init():
        out_ref[...] = y

    @pl.when(c != 0)
    def _acc():
        out_ref[...] = out_ref[...] + y


def _combine_body(shared_ref, g0_ref, g1_ref, w0_ref, w1_ref, out_ref):
    rr = jax.lax.broadcasted_iota(jnp.int32, (TC, TC), 0)
    cc = jax.lax.broadcasted_iota(jnp.int32, (TC, TC), 1)
    ident = (rr == cc).astype(F32)
    w0c = jax.lax.dot_general(
        ident, w0_ref[0], (((1,), (1,)), ((), ())),
        preferred_element_type=F32)
    w1c = jax.lax.dot_general(
        ident, w1_ref[0], (((1,), (1,)), ((), ())),
        preferred_element_type=F32)
    out_ref[...] = (shared_ref[...]
                    + w0c * g0_ref[...].astype(F32)
                    + w1c * g1_ref[...].astype(F32))


def _sc_mesh():
    return plsc.VectorSubcoreMesh(core_axis_name="c", subcore_axis_name="s")


def _sc_scatter(x3, s0m, s1m):
    @functools.partial(
        pl.kernel,
        mesh=_sc_mesh(),
        out_type=jax.ShapeDtypeStruct((NS, H), F32),
        scratch_types=[
            pltpu.VMEM((TPW,), jnp.int32),
            pltpu.VMEM((TPW,), jnp.int32),
            pltpu.VMEM((TPW, H), F32),
            pltpu.SemaphoreType.DMA,
        ],
    )
    def k(x_hbm, s0_hbm, s1_hbm, xg_hbm, idx0, idx1, rows, sem):
        wid = lax.axis_index("s") * 2 + lax.axis_index("c")
        base = wid * TPW
        pltpu.sync_copy(s0_hbm.at[wid], idx0)
        pltpu.sync_copy(s1_hbm.at[wid], idx1)
        pltpu.sync_copy(x_hbm.at[pl.ds(base, TPW)], rows)
        c0 = pltpu.async_copy(rows, xg_hbm.at[idx0], sem)
        c1 = pltpu.async_copy(rows, xg_hbm.at[idx1], sem)
        c0.wait()
        c1.wait()

    return k(x3, s0m, s1m)


def _sc_gather(yg3, s0m, s1m):
    @functools.partial(
        pl.kernel,
        mesh=_sc_mesh(),
        out_type=(jax.ShapeDtypeStruct((T, H), F32),
                  jax.ShapeDtypeStruct((T, H), F32)),
        scratch_types=[
            pltpu.VMEM((TPW,), jnp.int32),
            pltpu.VMEM((TPW, H), F32),
            pltpu.SemaphoreType.DMA,
        ],
    )
    def k(yg_hbm, s0_hbm, s1_hbm, g0_hbm, g1_hbm, idx, rows, sem):
        wid = lax.axis_index("s") * 2 + lax.axis_index("c")
        base = wid * TPW
        pltpu.sync_copy(s0_hbm.at[wid], idx)
        pltpu.async_copy(yg_hbm.at[idx], rows, sem).wait()
        pltpu.sync_copy(rows, g0_hbm.at[pl.ds(base, TPW)])
        pltpu.sync_copy(s1_hbm.at[wid], idx)
        pltpu.async_copy(yg_hbm.at[idx], rows, sem).wait()
        pltpu.sync_copy(rows, g1_hbm.at[pl.ds(base, TPW)])

    return k(yg3, s0m, s1m)


@jax.jit
def _run(x, router_weight, bias, gate_up_proj, down_proj,
         shared_gate_w, shared_up_w, shared_down_w):
    x_bf = x.astype(BF)
    bias_col = bias.reshape(E, 1)

    s0, s1, w0, w1, te = pl.pallas_call(
        _router_body,
        out_shape=(
            jax.ShapeDtypeStruct((1, T), jnp.int32),
            jax.ShapeDtypeStruct((1, T), jnp.int32),
            jax.ShapeDtypeStruct((1, T), F32),
            jax.ShapeDtypeStruct((1, T), F32),
            jax.ShapeDtypeStruct((1, NTILES), jnp.int32),
        ),
    )(x, router_weight, bias_col)

    s0m = s0.reshape(NW, TPW)
    s1m = s1.reshape(NW, TPW)

    xg = _sc_scatter(x, s0m, s1m)

    grid_spec = pltpu.PrefetchScalarGridSpec(
        num_scalar_prefetch=1,
        grid=(NTILES,),
        in_specs=[
            pl.BlockSpec((TS, H), lambda i, te_r: (i, 0)),
            pl.BlockSpec((1, 2 * I, H), lambda i, te_r: (te_r[i], 0, 0)),
            pl.BlockSpec((1, H, I), lambda i, te_r: (te_r[i], 0, 0)),
        ],
        out_specs=pl.BlockSpec((TS, H), lambda i, te_r: (i, 0)),
        scratch_shapes=[
            pltpu.VMEM((2 * I, H), BF),
            pltpu.VMEM((H, I), BF),
        ],
    )
    yg = pl.pallas_call(
        _ffn_body,
        grid_spec=grid_spec,
        out_shape=jax.ShapeDtypeStruct((NS, H), F32),
    )(te.reshape(NTILES), xg, gate_up_proj, down_proj)

    g0, g1 = _sc_gather(yg, s0m, s1m)

    shared = pl.pallas_call(
        _shared_body,
        grid=(NISC,),
        in_specs=[
            pl.BlockSpec((T, H), lambda c: (0, 0)),
            pl.BlockSpec((ISC, H), lambda c: (c, 0)),
            pl.BlockSpec((ISC, H), lambda c: (c, 0)),
            pl.BlockSpec((H, ISC), lambda c: (0, c)),
        ],
        out_specs=pl.BlockSpec((T, H), lambda c: (0, 0)),
        out_shape=jax.ShapeDtypeStruct((T, H), F32),
    )(x_bf, shared_gate_w, shared_up_w, shared_down_w)

    out = pl.pallas_call(
        _combine_body,
        grid=(NTC,),
        in_specs=[
            pl.BlockSpec((TC, H), lambda t: (t, 0)),
            pl.BlockSpec((TC, H), lambda t: (t, 0)),
            pl.BlockSpec((TC, H), lambda t: (t, 0)),
            pl.BlockSpec((1, 1, TC), lambda t: (0, 0, t)),
            pl.BlockSpec((1, 1, TC), lambda t: (0, 0, t)),
        ],
        out_specs=pl.BlockSpec((TC, H), lambda t: (t, 0)),
        out_shape=jax.ShapeDtypeStruct((T, H), F32),
    )(shared, g0, g1, w0.reshape(1, 1, T), w1.reshape(1, 1, T))

    return out


def kernel(hidden_states, router_weight, e_score_correction_bias,
           gate_up_proj, down_proj, shared_gate_w, shared_up_w,
           shared_down_w):
    B, S, Hd = hidden_states.shape
    x = hidden_states.reshape(-1, Hd)
    out = _run(x, router_weight, e_score_correction_bias, gate_up_proj,
               down_proj, shared_gate_w, shared_up_w, shared_down_w)
    return out.reshape(B, S, Hd)


# speedup vs baseline: 1.0954x; 1.0156x over previous
import functools

import jax
import jax.numpy as jnp
from jax import lax
from jax.experimental import pallas as pl
from jax.experimental.pallas import tpu as pltpu
from jax.experimental.pallas import tpu_sc as plsc

H = 1024
E = 8
I = 1024
IS = 2048
N_GROUP = 4
GSIZE = E // N_GROUP
TOPK_GROUP = 2
TOP_K = 2
ROUTED_SCALING = 2.5

T = 2048
TS = 256
NTILES = 24
NS = NTILES * TS
ISC = 256
NISC = IS // ISC
TC = 256
NTC = T // TC

NW = 32
TPW = T // NW

BF = jnp.bfloat16
F32 = jnp.float32


def _router_body(x_ref, wr_ref, bias_ref, s0_ref, s1_ref,
                 w0_ref, w1_ref, te_ref, ntu_ref):
    logits = jax.lax.dot_general(
        wr_ref[...], x_ref[...], (((1,), (1,)), ((), ())),
        preferred_element_type=F32)
    scores = jax.nn.sigmoid(logits)
    choice = scores + bias_ref[...]

    ch = [choice[e:e + 1, :] for e in range(E)]
    sc = [scores[e:e + 1, :] for e in range(E)]

    gs = [ch[2 * j] + ch[2 * j + 1] for j in range(N_GROUP)]
    gmask = []
    for j in range(N_GROUP):
        rank = jnp.zeros_like(gs[j])
        for k in range(N_GROUP):
            if k == j:
                continue
            beats = (gs[k] >= gs[j]) if k < j else (gs[k] > gs[j])
            rank = rank + beats.astype(F32)
        gmask.append(rank < float(TOPK_GROUP))

    ms = [jnp.where(gmask[e // GSIZE], ch[e], 0.0) for e in range(E)]

    sel = []
    for e in range(E):
        rank = jnp.zeros_like(ms[e])
        for f in range(E):
            if f == e:
                continue
            beats = (ms[f] >= ms[e]) if f < e else (ms[f] > ms[e])
            rank = rank + beats.astype(F32)
        sel.append((rank < float(TOP_K)).astype(F32))

    w = [sel[e] * sc[e] for e in range(E)]
    denom = w[0]
    for e in range(1, E):
        denom = denom + w[e]
    denom = denom + 1e-20
    comb = [(w[e] / denom) * ROUTED_SCALING for e in range(E)]

    selmat = jnp.concatenate(sel, axis=0)
    r = jax.lax.broadcasted_iota(jnp.int32, (T, T), 0)
    c = jax.lax.broadcasted_iota(jnp.int32, (T, T), 1)
    causal = (r <= c).astype(F32)
    posinc = jax.lax.dot_general(
        selmat, causal, (((1,), (0,)), ((), ())),
        preferred_element_type=F32)

    cnt = [jnp.sum(sel[e], axis=1, keepdims=True) for e in range(E)]
    caps = [jnp.floor((cnt[e] + (TS - 1)) / TS) * TS for e in range(E)]
    bases = []
    run = jnp.zeros_like(cnt[0])
    for e in range(E):
        bases.append(run)
        run = run + caps[e]

    slot = [bases[e] + posinc[e:e + 1, :] - sel[e] for e in range(E)]

    nb = jnp.ones_like(sel[0])
    s0 = jnp.zeros_like(sel[0])
    s1 = jnp.zeros_like(sel[0])
    w0 = jnp.zeros_like(sel[0])
    w1 = jnp.zeros_like(sel[0])
    for e in range(E):
        fm = sel[e] * nb
        nb = nb * (1.0 - sel[e])
        sm = sel[e] - fm
        s0 = s0 + fm * slot[e]
        s1 = s1 + sm * slot[e]
        w0 = w0 + fm * comb[e]
        w1 = w1 + sm * comb[e]
    s0_ref[...] = s0.astype(jnp.int32)
    s1_ref[...] = s1.astype(jnp.int32)

    ident = (r == c).astype(F32)
    w0_ref[...] = jax.lax.dot_general(
        ident, w0, (((1,), (1,)), ((), ())), preferred_element_type=F32)
    w1_ref[...] = jax.lax.dot_general(
        ident, w1, (((1,), (1,)), ((), ())), preferred_element_type=F32)

    jt = jax.lax.broadcasted_iota(jnp.int32, (1, NTILES), 1).astype(F32) * TS
    acc = jnp.zeros_like(jt)
    for e in range(E):
        acc = acc + (jt >= (bases[e] + caps[e])).astype(F32)
    te_ref[...] = jnp.minimum(acc, float(E - 1)).astype(jnp.int32)
    ntu_ref[...] = (run / float(TS)).astype(jnp.int32)


def _ffn_body(te_ref, ntu_ref, xg_ref, gu_ref, d_ref, out_ref, gu_bf, d_bf):
    i = pl.program_id(0)

    @pl.when(i < ntu_ref[0])
    def _work():
        iprev = jnp.maximum(i - 1, 0)
        recast = (i == 0) | (te_ref[i] != te_ref[iprev])

        @pl.when(recast)
        def _cast():
            gu_bf[...] = gu_ref[0].astype(BF)
            d_bf[...] = d_ref[0].astype(BF)

        xb = xg_ref[...].astype(BF)
        gate = jax.lax.dot_general(
            xb, gu_bf[:I, :], (((1,), (1,)), ((), ())),
            preferred_element_type=F32)
        up = jax.lax.dot_general(
            xb, gu_bf[I:, :], (((1,), (1,)), ((), ())),
            preferred_element_type=F32)
        h = ((gate * jax.nn.sigmoid(gate)) * up).astype(BF)
        y = jax.lax.dot_general(
            h, d_bf[...], (((1,), (1,)), ((), ())),
            preferred_element_type=F32)
        out_ref[...] = y


def _shared_body(x_ref, gw_ref, uw_ref, dw_ref, g0_ref, g1_ref,
                 w0_ref, w1_ref, out_ref):
    c = pl.program_id(0)
    xb = x_ref[...]
    gate = jax.lax.dot_general(
        xb, gw_ref[...].astype(BF), (((1,), (1,)), ((), ())),
        preferred_element_type=F32)
    up = jax.lax.dot_general(
        xb, uw_ref[...].astype(BF), (((1,), (1,)), ((), ())),
        preferred_element_type=F32)
    h = ((gate * jax.nn.sigmoid(gate)) * up).astype(BF)
    y = jax.lax.dot_general(
        h, dw_ref[...].astype(BF), (((1,), (1,)), ((), ())),
        preferred_element_type=F32)

    @pl.when(c == 0)
    def _init():
        out_ref[...] = y

    @pl.when((c != 0) & (c != NISC - 1))
    def _acc():
        out_ref[...] = out_ref[...] + y

    @pl.when(c == NISC - 1)
    def _fin():
        out_ref[...] = (out_ref[...] + y
                        + w0_ref[...] * g0_ref[...]
                        + w1_ref[...] * g1_ref[...])


def _sc_mesh():
    return plsc.VectorSubcoreMesh(core_axis_name="c", subcore_axis_name="s")


def _sc_scatter(x3, s0m, s1m):
    @functools.partial(
        pl.kernel,
        mesh=_sc_mesh(),
        out_type=jax.ShapeDtypeStruct((NS, H), F32),
        scratch_types=[
            pltpu.VMEM((TPW,), jnp.int32),
            pltpu.VMEM((TPW,), jnp.int32),
            pltpu.VMEM((TPW, H), F32),
            pltpu.SemaphoreType.DMA,
        ],
    )
    def k(x_hbm, s0_hbm, s1_hbm, xg_hbm, idx0, idx1, rows, sem):
        wid = lax.axis_index("s") * 2 + lax.axis_index("c")
        base = wid * TPW
        pltpu.sync_copy(s0_hbm.at[wid], idx0)
        pltpu.sync_copy(s1_hbm.at[wid], idx1)
        pltpu.sync_copy(x_hbm.at[pl.ds(base, TPW)], rows)
        c0 = pltpu.async_copy(rows, xg_hbm.at[idx0], sem)
        c1 = pltpu.async_copy(rows, xg_hbm.at[idx1], sem)
        c0.wait()
        c1.wait()

    return k(x3, s0m, s1m)


def _sc_gather(yg3, s0m, s1m):
    @functools.partial(
        pl.kernel,
        mesh=_sc_mesh(),
        out_type=(jax.ShapeDtypeStruct((T, H), F32),
                  jax.ShapeDtypeStruct((T, H), F32)),
        scratch_types=[
            pltpu.VMEM((TPW,), jnp.int32),
            pltpu.VMEM((TPW, H), F32),
            pltpu.SemaphoreType.DMA,
        ],
    )
    def k(yg_hbm, s0_hbm, s1_hbm, g0_hbm, g1_hbm, idx, rows, sem):
        wid = lax.axis_index("s") * 2 + lax.axis_index("c")
        base = wid * TPW
        pltpu.sync_copy(s0_hbm.at[wid], idx)
        pltpu.async_copy(yg_hbm.at[idx], rows, sem).wait()
        pltpu.sync_copy(rows, g0_hbm.at[pl.ds(base, TPW)])
        pltpu.sync_copy(s1_hbm.at[wid], idx)
        pltpu.async_copy(yg_hbm.at[idx], rows, sem).wait()
        pltpu.sync_copy(rows, g1_hbm.at[pl.ds(base, TPW)])

    return k(yg3, s0m, s1m)


@jax.jit
def _run(x, router_weight, bias, gate_up_proj, down_proj,
         shared_gate_w, shared_up_w, shared_down_w):
    x_bf = x.astype(BF)
    bias_col = bias.reshape(E, 1)

    s0, s1, w0c, w1c, te, ntu = pl.pallas_call(
        _router_body,
        out_shape=(
            jax.ShapeDtypeStruct((1, T), jnp.int32),
            jax.ShapeDtypeStruct((1, T), jnp.int32),
            jax.ShapeDtypeStruct((T, 1), F32),
            jax.ShapeDtypeStruct((T, 1), F32),
            jax.ShapeDtypeStruct((1, NTILES), jnp.int32),
            jax.ShapeDtypeStruct((1, 1), jnp.int32),
        ),
    )(x, router_weight, bias_col)

    s0m = s0.reshape(NW, TPW)
    s1m = s1.reshape(NW, TPW)

    xg = _sc_scatter(x, s0m, s1m)

    grid_spec = pltpu.PrefetchScalarGridSpec(
        num_scalar_prefetch=2,
        grid=(NTILES,),
        in_specs=[
            pl.BlockSpec((TS, H), lambda i, te_r, nt_r: (i, 0)),
            pl.BlockSpec((1, 2 * I, H), lambda i, te_r, nt_r: (te_r[i], 0, 0)),
            pl.BlockSpec((1, H, I), lambda i, te_r, nt_r: (te_r[i], 0, 0)),
        ],
        out_specs=pl.BlockSpec((TS, H), lambda i, te_r, nt_r: (i, 0)),
        scratch_shapes=[
            pltpu.VMEM((2 * I, H), BF),
            pltpu.VMEM((H, I), BF),
        ],
    )
    yg = pl.pallas_call(
        _ffn_body,
        grid_spec=grid_spec,
        out_shape=jax.ShapeDtypeStruct((NS, H), F32),
    )(te.reshape(NTILES), ntu.reshape(1), xg, gate_up_proj, down_proj)

    g0, g1 = _sc_gather(yg, s0m, s1m)

    out = pl.pallas_call(
        _shared_body,
        grid=(NISC,),
        in_specs=[
            pl.BlockSpec((T, H), lambda c: (0, 0)),
            pl.BlockSpec((ISC, H), lambda c: (c, 0)),
            pl.BlockSpec((ISC, H), lambda c: (c, 0)),
            pl.BlockSpec((H, ISC), lambda c: (0, c)),
            pl.BlockSpec((T, H), lambda c: (0, 0)),
            pl.BlockSpec((T, H), lambda c: (0, 0)),
            pl.BlockSpec((T, 1), lambda c: (0, 0)),
            pl.BlockSpec((T, 1), lambda c: (0, 0)),
        ],
        out_specs=pl.BlockSpec((T, H), lambda c: (0, 0)),
        out_shape=jax.ShapeDtypeStruct((T, H), F32),
    )(x_bf, shared_gate_w, shared_up_w, shared_down_w, g0, g1, w0c, w1c)

    return out


def kernel(hidden_states, router_weight, e_score_correction_bias,
           gate_up_proj, down_proj, shared_gate_w, shared_up_w,
           shared_down_w):
    B, S, Hd = hidden_states.shape
    x = hidden_states.reshape(-1, Hd)
    out = _run(x, router_weight, e_score_correction_bias, gate_up_proj,
               down_proj, shared_gate_w, shared_up_w, shared_down_w)
    return out.reshape(B, S, Hd)
